# baseline (speedup 1.0000x reference)
import functools

import jax
import jax.numpy as jnp
from jax import lax
from jax.experimental import pallas as pl
from jax.experimental.pallas import tpu as pltpu
from jax.experimental.pallas import tpu_sc as plsc

_B = 16384
_D = 32
_ROWS = 1000000
_QW = 256
_NQ = _ROWS * _D // _QW
_INFO = plsc.get_sparse_core_info()
_NC = _INFO.num_cores
_NS = _INFO.num_subcores
_L = _INFO.num_lanes
_NW = _NC * _NS
_BPW = _B // _NW
_C = 64
_NCHUNK = _BPW // _C
_GPC = _C // _L


def _rsqrt(x):
    i = plsc.bitcast(x, jnp.int32)
    i = 0x5F3759DF - lax.shift_right_logical(i, 1)
    y = plsc.bitcast(i, jnp.float32)
    for _ in range(3):
        y = y * (1.5 - 0.5 * x * y * y)
    return y


def _renorm_scale(sumsq):
    r = _rsqrt(sumsq)
    norm = sumsq * r
    inv = 1.0 / (norm + 1e-7)
    return jnp.where(norm > 1.0, inv, jnp.ones_like(norm))


_MESH = plsc.VectorSubcoreMesh(core_axis_name="c", subcore_axis_name="s")


@functools.partial(
    pl.kernel,
    mesh=_MESH,
    compiler_params=pltpu.CompilerParams(
        needs_layout_passes=False, use_tc_tiling_on_sc=True),
    out_type=jax.ShapeDtypeStruct((_B,), jnp.float32),
    scratch_types=[
        pltpu.VMEM((_BPW,), jnp.int32),
        pltpu.VMEM((_BPW,), jnp.int32),
        pltpu.VMEM((_BPW,), jnp.int32),
        pltpu.VMEM((_BPW,), jnp.int32),
        pltpu.VMEM((_C, _QW), jnp.float32),
        pltpu.VMEM((_C, _QW), jnp.float32),
        pltpu.VMEM((_C, _QW), jnp.float32),
        pltpu.VMEM((_C, _QW), jnp.float32),
        pltpu.VMEM((_BPW,), jnp.float32),
        pltpu.SemaphoreType.DMA,
        pltpu.SemaphoreType.DMA,
    ],
)
def _mf_kernel(users_hbm, items_hbm, utab_hbm, itab_hbm, out_hbm,
               uidx_s, iidx_s, uq_v, iq_v,
               au_v, av_v, bu_v, bv_v, out_v, sem_a, sem_b):
    wid = lax.axis_index("s") * _NC + lax.axis_index("c")
    base = wid * _BPW
    lane = lax.iota(jnp.int32, _L)

    pltpu.sync_copy(users_hbm.at[pl.ds(base, _BPW)], uidx_s)
    pltpu.sync_copy(items_hbm.at[pl.ds(base, _BPW)], iidx_s)
    for k in range(_BPW // _L):
        sl = pl.ds(k * _L, _L)
        uq_v[sl] = lax.shift_right_logical(uidx_s[sl], 3)
        iq_v[sl] = lax.shift_right_logical(iidx_s[sl], 3)

    def fire(c, bufu, bufv, sem):
        sl = pl.ds(c * _C, _C)
        pltpu.async_copy(utab_hbm.at[uq_v.at[sl]], bufu, sem)
        pltpu.async_copy(itab_hbm.at[iq_v.at[sl]], bufv, sem)

    def wait_pair(bufu, bufv, sem):
        pltpu.make_async_copy(utab_hbm.at[pl.ds(0, _C)], bufu, sem).wait()
        pltpu.make_async_copy(itab_hbm.at[pl.ds(0, _C)], bufv, sem).wait()

    def compute(c, bufu, bufv):
        def group_body(g, carry):
            ex0 = c * _C + g * _L
            uu = jnp.zeros((_L,), jnp.float32)
            vv = jnp.zeros((_L,), jnp.float32)
            uv = jnp.zeros((_L,), jnp.float32)
            su_vec = (uidx_s[pl.ds(ex0, _L)] & 7) * _D
            sv_vec = (iidx_s[pl.ds(ex0, _L)] & 7) * _D
            for e in range(_L):
                r = g * _L + e
                su = su_vec[e]
                sv = sv_vec[e]
                u_lo = bufu[r, pl.ds(su, _L)]
                u_hi = bufu[r, pl.ds(su + _L, _L)]
                v_lo = bufv[r, pl.ds(sv, _L)]
                v_hi = bufv[r, pl.ds(sv + _L, _L)]
                p_uu = u_lo * u_lo + u_hi * u_hi
                p_vv = v_lo * v_lo + v_hi * v_hi
                p_uv = u_lo * v_lo + u_hi * v_hi
                m = lane == e
                uu = jnp.where(m, jnp.sum(p_uu), uu)
                vv = jnp.where(m, jnp.sum(p_vv), vv)
                uv = jnp.where(m, jnp.sum(p_uv), uv)
            scale = _renorm_scale(uu) * _renorm_scale(vv)
            out_v[pl.ds(ex0, _L)] = uv * scale
            return carry

        lax.fori_loop(0, _GPC, group_body, 0)

    fire(0, au_v, av_v, sem_a)

    def body(i, carry):
        c0 = 2 * i
        fire(c0 + 1, bu_v, bv_v, sem_b)
        wait_pair(au_v, av_v, sem_a)
        compute(c0, au_v, av_v)

        @pl.when(i < _NCHUNK // 2 - 1)
        def _():
            fire(c0 + 2, au_v, av_v, sem_a)

        wait_pair(bu_v, bv_v, sem_b)
        compute(c0 + 1, bu_v, bv_v)
        return carry

    lax.fori_loop(0, _NCHUNK // 2, body, 0)

    pltpu.sync_copy(out_v, out_hbm.at[pl.ds(base, _BPW)])


def kernel(users, items, user_table, item_table):
    utab2 = user_table.reshape(_NQ, _QW)
    itab2 = item_table.reshape(_NQ, _QW)
    return _mf_kernel(users.astype(jnp.int32), items.astype(jnp.int32),
                      utab2, itab2)


# speedup vs baseline: 2.3543x; 2.3543x over previous
import functools

import jax
import jax.numpy as jnp
from jax import lax
from jax.experimental import pallas as pl
from jax.experimental.pallas import tpu as pltpu
from jax.experimental.pallas import tpu_sc as plsc

_B = 16384
_D = 32
_ROWS = 1000000
_SUB = 8
_NT = _ROWS // _SUB
_INFO = plsc.get_sparse_core_info()
_NC = _INFO.num_cores
_NS = _INFO.num_subcores
_L = _INFO.num_lanes
_NW = _NC * _NS
_BPW = _B // _NW
_C = _L
_NCHUNK = _BPW // _C


def _rsqrt(x):
    i = plsc.bitcast(x, jnp.int32)
    i = 0x5F3759DF - lax.shift_right_logical(i, 1)
    y = plsc.bitcast(i, jnp.float32)
    for _ in range(3):
        y = y * (1.5 - 0.5 * x * y * y)
    return y


def _renorm_scale(sumsq):
    r = _rsqrt(sumsq)
    norm = sumsq * r
    inv = 1.0 / (norm + 1e-7)
    return jnp.where(norm > 1.0, inv, jnp.ones_like(norm))


_MESH = plsc.VectorSubcoreMesh(core_axis_name="c", subcore_axis_name="s")


@functools.partial(
    pl.kernel,
    mesh=_MESH,
    compiler_params=pltpu.CompilerParams(
        needs_layout_passes=False, use_tc_tiling_on_sc=True),
    out_type=jax.ShapeDtypeStruct((_B,), jnp.float32),
    scratch_types=[
        pltpu.VMEM((_BPW,), jnp.int32),
        pltpu.VMEM((_BPW,), jnp.int32),
        pltpu.VMEM((_C, _SUB, _D), jnp.float32),
        pltpu.VMEM((_C, _SUB, _D), jnp.float32),
        pltpu.VMEM((_C, _SUB, _D), jnp.float32),
        pltpu.VMEM((_C, _SUB, _D), jnp.float32),
        pltpu.VMEM((_BPW,), jnp.float32),
        pltpu.SemaphoreType.DMA,
        pltpu.SemaphoreType.DMA,
    ],
)
def _mf_kernel(users_hbm, items_hbm, utab_hbm, itab_hbm, out_hbm,
               uidx_s, iidx_s,
               au_v, av_v, bu_v, bv_v, out_v, sem_a, sem_b):
    wid = lax.axis_index("s") * _NC + lax.axis_index("c")
    base = wid * _BPW
    lane = lax.iota(jnp.int32, _L)

    pltpu.sync_copy(users_hbm.at[pl.ds(base, _BPW)], uidx_s)
    pltpu.sync_copy(items_hbm.at[pl.ds(base, _BPW)], iidx_s)

    def fire(c, bufu, bufv, sem):
        ex0 = c * _C
        tu_vec = lax.shift_right_logical(uidx_s[pl.ds(ex0, _L)], 3)
        ti_vec = lax.shift_right_logical(iidx_s[pl.ds(ex0, _L)], 3)
        for e in range(_C):
            pltpu.async_copy(utab_hbm.at[pl.ds(tu_vec[e], 1)],
                             bufu.at[pl.ds(e, 1)], sem)
            pltpu.async_copy(itab_hbm.at[pl.ds(ti_vec[e], 1)],
                             bufv.at[pl.ds(e, 1)], sem)

    def wait_pair(bufu, bufv, sem):
        pltpu.make_async_copy(utab_hbm.at[pl.ds(0, _C)], bufu, sem).wait()
        pltpu.make_async_copy(itab_hbm.at[pl.ds(0, _C)], bufv, sem).wait()

    def compute(c, bufu, bufv):
        ex0 = c * _C
        uu = jnp.zeros((_L,), jnp.float32)
        vv = jnp.zeros((_L,), jnp.float32)
        uv = jnp.zeros((_L,), jnp.float32)
        su_vec = uidx_s[pl.ds(ex0, _L)] & 7
        sv_vec = iidx_s[pl.ds(ex0, _L)] & 7
        for e in range(_C):
            su = su_vec[e]
            sv = sv_vec[e]
            u_lo = bufu[e, su, pl.ds(0, _L)]
            u_hi = bufu[e, su, pl.ds(_L, _L)]
            v_lo = bufv[e, sv, pl.ds(0, _L)]
            v_hi = bufv[e, sv, pl.ds(_L, _L)]
            p_uu = u_lo * u_lo + u_hi * u_hi
            p_vv = v_lo * v_lo + v_hi * v_hi
            p_uv = u_lo * v_lo + u_hi * v_hi
            m = lane == e
            uu = jnp.where(m, jnp.sum(p_uu), uu)
            vv = jnp.where(m, jnp.sum(p_vv), vv)
            uv = jnp.where(m, jnp.sum(p_uv), uv)
        scale = _renorm_scale(uu) * _renorm_scale(vv)
        out_v[pl.ds(ex0, _L)] = uv * scale

    fire(0, au_v, av_v, sem_a)

    def body(i, carry):
        c0 = 2 * i
        fire(c0 + 1, bu_v, bv_v, sem_b)
        wait_pair(au_v, av_v, sem_a)
        compute(c0, au_v, av_v)

        @pl.when(i < _NCHUNK // 2 - 1)
        def _():
            fire(c0 + 2, au_v, av_v, sem_a)

        wait_pair(bu_v, bv_v, sem_b)
        compute(c0 + 1, bu_v, bv_v)
        return carry

    lax.fori_loop(0, _NCHUNK // 2, body, 0)

    pltpu.sync_copy(out_v, out_hbm.at[pl.ds(base, _BPW)])


def kernel(users, items, user_table, item_table):
    utab3 = user_table.reshape(_NT, _SUB, _D)
    itab3 = item_table.reshape(_NT, _SUB, _D)
    return _mf_kernel(users.astype(jnp.int32), items.astype(jnp.int32),
                      utab3, itab3)
